# baseline (speedup 1.0000x reference)
import functools

import jax
import jax.numpy as jnp
from jax import lax
from jax.experimental import pallas as pl
from jax.experimental.pallas import tpu as pltpu
from jax.experimental.pallas import tpu_sc as plsc

E = 64
TOP_K = 8
D_MODEL = 768
D_FF = 384

T = 2048
A = T * TOP_K
BM = 256
BM_SHIFT = 8
NB = A // BM + E
PADN = NB * BM

TB = 256

NW = 32
AW = A // NW
GW = AW // 16
TW = T // NW
NG = A // 16

_SC_MESH = plsc.VectorSubcoreMesh(core_axis_name="c", subcore_axis_name="s",
                                  num_cores=2, num_subcores=16)
_SC_PARAMS = pltpu.CompilerParams(needs_layout_passes=False)


def _router_body(x_ref, gw_ref, exp_ref, rw_ref):
    x = x_ref[...]
    gw = gw_ref[...]
    logits = lax.dot_general(x, gw, (((1,), (1,)), ((), ())),
                             preferred_element_type=jnp.float32)
    m = jnp.max(logits, axis=-1, keepdims=True)
    p = jnp.exp(logits - m)
    p = p / jnp.sum(p, axis=-1, keepdims=True)

    iota = lax.broadcasted_iota(jnp.int32, (TB, E), 1)
    vals = []
    idxs = []
    for _ in range(TOP_K):
        mx = jnp.max(p, axis=-1, keepdims=True)
        idx = jnp.min(jnp.where(p == mx, iota, E), axis=-1, keepdims=True)
        vals.append(mx)
        idxs.append(idx)
        p = jnp.where(iota == idx, -1.0, p)
    v = jnp.concatenate(vals, axis=-1)
    ix = jnp.concatenate(idxs, axis=-1)
    v = v / jnp.sum(v, axis=-1, keepdims=True)
    exp_ref[...] = ix
    rw_ref[...] = v


def _router(flat, gate_w):
    return pl.pallas_call(
        _router_body,
        grid=(T // TB,),
        in_specs=[
            pl.BlockSpec((TB, D_MODEL), lambda i: (i, 0)),
            pl.BlockSpec((E, D_MODEL), lambda i: (0, 0)),
        ],
        out_specs=[
            pl.BlockSpec((TB, TOP_K), lambda i: (i, 0)),
            pl.BlockSpec((TB, TOP_K), lambda i: (i, 0)),
        ],
        out_shape=[
            jax.ShapeDtypeStruct((T, TOP_K), jnp.int32),
            jax.ShapeDtypeStruct((T, TOP_K), jnp.float32),
        ],
    )(flat, gate_w)


@functools.partial(
    pl.kernel,
    out_type=[
        jax.ShapeDtypeStruct((PADN, D_MODEL), jnp.float32),
        jax.ShapeDtypeStruct((A,), jnp.int32),
        jax.ShapeDtypeStruct((NB,), jnp.int32),
        jax.ShapeDtypeStruct((16,), jnp.int32),
    ],
    mesh=_SC_MESH,
    compiler_params=_SC_PARAMS,
    scratch_types=[
        pltpu.VMEM((A,), jnp.int32),
        pltpu.VMEM((E,), jnp.int32),
        pltpu.VMEM((AW,), jnp.int32),
        pltpu.VMEM((AW,), jnp.int32),
        pltpu.VMEM((E,), jnp.int32),
        pltpu.VMEM((NB + 16,), jnp.int32),
        pltpu.VMEM((NB,), jnp.int32),
        pltpu.VMEM((TW, D_MODEL), jnp.float32),
        [pltpu.VMEM((TW,), jnp.int32) for _ in range(TOP_K)],
        pltpu.VMEM((16,), jnp.int32),
        pltpu.SemaphoreType.DMA,
    ],
)
def _shuffle(e_hbm, flat_hbm, xs_hbm, pos_hbm, be_hbm, nl_hbm,
             ec, hist, lr, posc, sr, marks, bemap, tokrows, iks, stg, ss):
    wid = lax.axis_index("s") * 2 + lax.axis_index("c")
    ab = wid * AW

    pltpu.sync_copy(e_hbm, ec)
    for c in range(E // 16):
        hist[pl.ds(c * 16, 16)] = jnp.zeros((16,), jnp.int32)

    ones = jnp.ones((16,), jnp.int32)
    lane = lax.iota(jnp.int32, 16)

    def hist_pass(j, _):
        ev = ec[pl.ds(j * 16, 16)]
        cnt, _last = plsc.scan_count(ev)
        prior = plsc.load_gather(hist, [ev])

        @pl.when((j >= wid * GW) & (j < (wid + 1) * GW))
        def _():
            lr[pl.ds((j - wid * GW) * 16, 16)] = prior + cnt - 1

        plsc.addupdate_scatter(hist, [ev], ones)
        return 0

    lax.fori_loop(0, NG, hist_pass, 0, unroll=4)

    for c in range((NB + 16) // 16):
        marks[pl.ds(c * 16, 16)] = jnp.zeros((16,), jnp.int32)
    carry = jnp.zeros((), jnp.int32)
    for c in range(E // 16):
        h = hist[pl.ds(c * 16, 16)]
        nb = (h + (BM - 1)) >> BM_SHIFT
        s = plsc.cumsum(nb)
        ex = s - nb + carry
        sr[pl.ds(c * 16, 16)] = ex * BM
        plsc.addupdate_scatter(marks, [ex + nb], ones)
        carry = carry + jnp.sum(nb)
    nlive = carry

    bcarry = jnp.zeros((), jnp.int32)
    for c in range(NB // 16):
        m = marks[pl.ds(c * 16, 16)]
        cs = plsc.cumsum(m)
        bemap[pl.ds(c * 16, 16)] = cs + bcarry
        bcarry = bcarry + jnp.sum(m)

    @pl.when(wid == 0)
    def _():
        pltpu.sync_copy(bemap, be_hbm)
        stg[...] = jnp.full((16,), nlive, jnp.int32)
        pltpu.sync_copy(stg, nl_hbm)

    def pos_pass(j, _):
        ev = ec[pl.ds((wid * GW + j) * 16, 16)]
        grank = lr[pl.ds(j * 16, 16)]
        posc[pl.ds(j * 16, 16)] = plsc.load_gather(sr, [ev]) + grank
        return 0

    lax.fori_loop(0, GW, pos_pass, 0, unroll=4)

    pltpu.sync_copy(flat_hbm.at[pl.ds(wid * TW, TW)], tokrows)
    for k in range(TOP_K):
        for c in range(TW // 16):
            iks[k][pl.ds(c * 16, 16)] = plsc.load_gather(
                posc, [lane * TOP_K + c * 16 * TOP_K + k])
    for k in range(TOP_K):
        pltpu.async_copy(tokrows, xs_hbm.at[iks[k]], ss)
    for k in range(TOP_K):
        pltpu.make_async_copy(tokrows, xs_hbm.at[iks[k]], ss).wait()
    pltpu.sync_copy(posc, pos_hbm.at[pl.ds(ab, AW)])


def _gemm_body(blk_exp_ref, nlive_ref, x_ref, wgu_ref, wd_ref, y_ref):
    b = pl.program_id(0)

    @pl.when(b < nlive_ref[0])
    def _():
        x = x_ref[...]
        gu = lax.dot_general(x, wgu_ref[0], (((1,), (0,)), ((), ())),
                             preferred_element_type=jnp.float32)
        g = gu[:, :D_FF]
        u = gu[:, D_FF:]
        h = g * jax.nn.sigmoid(g) * u
        y_ref[...] = lax.dot_general(h, wd_ref[0], (((1,), (0,)), ((), ())),
                                     preferred_element_type=jnp.float32)


def _grouped_gemm(x_sorted, w_gate_up, w_down, blk_exp, nlive):
    def live_b(b, blk_exp_ref, nlive_ref):
        return jnp.minimum(b, nlive_ref[0] - 1)

    grid_spec = pltpu.PrefetchScalarGridSpec(
        num_scalar_prefetch=2,
        grid=(NB,),
        in_specs=[
            pl.BlockSpec((BM, D_MODEL), lambda b, be, nl: (live_b(b, be, nl), 0)),
            pl.BlockSpec((1, D_MODEL, 2 * D_FF),
                         lambda b, be, nl: (be[live_b(b, be, nl)], 0, 0)),
            pl.BlockSpec((1, D_FF, D_MODEL),
                         lambda b, be, nl: (be[live_b(b, be, nl)], 0, 0)),
        ],
        out_specs=pl.BlockSpec((BM, D_MODEL), lambda b, be, nl: (live_b(b, be, nl), 0)),
    )
    return pl.pallas_call(
        _gemm_body,
        grid_spec=grid_spec,
        out_shape=jax.ShapeDtypeStruct((PADN, D_MODEL), jnp.float32),
    )(blk_exp, nlive, x_sorted, w_gate_up, w_down)


@functools.partial(
    pl.kernel,
    out_type=jax.ShapeDtypeStruct((T, D_MODEL), jnp.float32),
    mesh=_SC_MESH,
    compiler_params=_SC_PARAMS,
    scratch_types=[
        pltpu.VMEM((AW,), jnp.int32),
        pltpu.VMEM((AW,), jnp.float32),
        [pltpu.VMEM((16,), jnp.int32) for _ in range(4)],
        [pltpu.VMEM((16, D_MODEL), jnp.float32) for _ in range(4)],
        pltpu.VMEM((TW, D_MODEL), jnp.float32),
        [pltpu.SemaphoreType.DMA for _ in range(4)],
    ],
)
def _combine(y_hbm, rw_hbm, pos_hbm, out_hbm, posc, wc, idx, rows, outr, sem):
    wid = lax.axis_index("s") * 2 + lax.axis_index("c")
    ab = wid * AW

    pltpu.sync_copy(pos_hbm.at[pl.ds(ab, AW)], posc)
    pltpu.sync_copy(rw_hbm.at[pl.ds(ab, AW)], wc)

    NBUF = 4

    def issue(p, j):
        idx[p][...] = posc[pl.ds(j * 16, 16)]
        pltpu.async_copy(y_hbm.at[idx[p]], rows[p], sem[p])

    def drain(p):
        pltpu.make_async_copy(y_hbm.at[idx[p]], rows[p], sem[p]).wait()

    def compute(p, j):
        wk = [plsc.load_gather(wc, [jnp.full((16,), j * 16 + k, jnp.int32)])
              for k in range(16)]
        for c in range(D_MODEL // 16):
            for lt in range(2):
                r = lambda k: rows[p][lt * 8 + k, pl.ds(c * 16, 16)]
                w = lambda k: wk[lt * 8 + k]
                acc_a = w(0) * r(0) + w(1) * r(1)
                acc_a = acc_a + w(2) * r(2) + w(3) * r(3)
                acc_b = w(4) * r(4) + w(5) * r(5)
                acc_b = acc_b + w(6) * r(6) + w(7) * r(7)
                outr[j * 2 + lt, pl.ds(c * 16, 16)] = acc_a + acc_b

    for p in range(NBUF):
        issue(p, p)

    def quad(i, _):
        j = i * NBUF
        for p in range(NBUF):
            drain(p)
            compute(p, j + p)

            @pl.when(j + p + NBUF < GW)
            def _():
                issue(p, j + p + NBUF)

        return 0

    lax.fori_loop(0, GW // NBUF, quad, 0)
    pltpu.sync_copy(outr, out_hbm.at[pl.ds(wid * TW, TW)])


def kernel(hidden_states, gate_w, w_gate_up, w_down):
    orig_shape = hidden_states.shape
    flat = hidden_states.reshape(T, D_MODEL)

    experts, rw = _router(flat, gate_w)
    x_sorted, pos, blk_exp, nlive = _shuffle(experts.reshape(A), flat)
    y = _grouped_gemm(x_sorted, w_gate_up, w_down, blk_exp, nlive)
    final = _combine(y, rw.reshape(A), pos)
    return final.reshape(orig_shape)


# speedup vs baseline: 1.0573x; 1.0573x over previous
import functools

import jax
import jax.numpy as jnp
from jax import lax
from jax.experimental import pallas as pl
from jax.experimental.pallas import tpu as pltpu
from jax.experimental.pallas import tpu_sc as plsc

E = 64
TOP_K = 8
D_MODEL = 768
D_FF = 384

T = 2048
A = T * TOP_K
BM = 256
BM_SHIFT = 8
NB = A // BM + E
PADN = NB * BM

TB = 256

NW = 32
AW = A // NW
GW = AW // 16
TW = T // NW
NG = A // 16

_SC_MESH = plsc.VectorSubcoreMesh(core_axis_name="c", subcore_axis_name="s",
                                  num_cores=2, num_subcores=16)
_SC_PARAMS = pltpu.CompilerParams(needs_layout_passes=False)


def _router_body(x_ref, gw_ref, exp_ref, rw_ref):
    x = x_ref[...]
    gw = gw_ref[...]
    logits = lax.dot_general(x, gw, (((1,), (1,)), ((), ())),
                             preferred_element_type=jnp.float32)
    m = jnp.max(logits, axis=-1, keepdims=True)
    p = jnp.exp(logits - m)
    p = p / jnp.sum(p, axis=-1, keepdims=True)

    iota = lax.broadcasted_iota(jnp.int32, (TB, E), 1)
    vals = []
    idxs = []
    for _ in range(TOP_K):
        mx = jnp.max(p, axis=-1, keepdims=True)
        idx = jnp.min(jnp.where(p == mx, iota, E), axis=-1, keepdims=True)
        vals.append(mx)
        idxs.append(idx)
        p = jnp.where(iota == idx, -1.0, p)
    v = jnp.concatenate(vals, axis=-1)
    ix = jnp.concatenate(idxs, axis=-1)
    v = v / jnp.sum(v, axis=-1, keepdims=True)
    exp_ref[...] = ix
    rw_ref[...] = v


def _router(flat, gate_w):
    return pl.pallas_call(
        _router_body,
        grid=(T // TB,),
        in_specs=[
            pl.BlockSpec((TB, D_MODEL), lambda i: (i, 0)),
            pl.BlockSpec((E, D_MODEL), lambda i: (0, 0)),
        ],
        out_specs=[
            pl.BlockSpec((TB, TOP_K), lambda i: (i, 0)),
            pl.BlockSpec((TB, TOP_K), lambda i: (i, 0)),
        ],
        out_shape=[
            jax.ShapeDtypeStruct((T, TOP_K), jnp.int32),
            jax.ShapeDtypeStruct((T, TOP_K), jnp.float32),
        ],
    )(flat, gate_w)


@functools.partial(
    pl.kernel,
    out_type=[
        jax.ShapeDtypeStruct((PADN, D_MODEL), jnp.float32),
        jax.ShapeDtypeStruct((A,), jnp.int32),
        jax.ShapeDtypeStruct((NB,), jnp.int32),
        jax.ShapeDtypeStruct((16,), jnp.int32),
    ],
    mesh=_SC_MESH,
    compiler_params=_SC_PARAMS,
    scratch_types=[
        pltpu.VMEM((A,), jnp.int32),
        pltpu.VMEM((E,), jnp.int32),
        pltpu.VMEM((AW,), jnp.int32),
        pltpu.VMEM((AW,), jnp.int32),
        pltpu.VMEM((E,), jnp.int32),
        pltpu.VMEM((NB + 16,), jnp.int32),
        pltpu.VMEM((NB,), jnp.int32),
        pltpu.VMEM((TW, D_MODEL), jnp.float32),
        [pltpu.VMEM((TW,), jnp.int32) for _ in range(TOP_K)],
        pltpu.VMEM((16,), jnp.int32),
        pltpu.SemaphoreType.DMA,
    ],
)
def _shuffle(e_hbm, flat_hbm, xs_hbm, pos_hbm, be_hbm, nl_hbm,
             ec, hist, lr, posc, sr, marks, bemap, tokrows, iks, stg, ss):
    wid = lax.axis_index("s") * 2 + lax.axis_index("c")
    ab = wid * AW

    pltpu.sync_copy(e_hbm, ec)
    for c in range(E // 16):
        hist[pl.ds(c * 16, 16)] = jnp.zeros((16,), jnp.int32)

    ones = jnp.ones((16,), jnp.int32)
    lane = lax.iota(jnp.int32, 16)

    def hist_pass(j, _):
        ev = ec[pl.ds(j * 16, 16)]
        cnt, _last = plsc.scan_count(ev)
        prior = plsc.load_gather(hist, [ev])

        @pl.when((j >= wid * GW) & (j < (wid + 1) * GW))
        def _():
            lr[pl.ds((j - wid * GW) * 16, 16)] = prior + cnt - 1

        plsc.addupdate_scatter(hist, [ev], ones)
        return 0

    lax.fori_loop(0, NG, hist_pass, 0, unroll=4)

    for c in range((NB + 16) // 16):
        marks[pl.ds(c * 16, 16)] = jnp.zeros((16,), jnp.int32)
    carry = jnp.zeros((), jnp.int32)
    for c in range(E // 16):
        h = hist[pl.ds(c * 16, 16)]
        nb = (h + (BM - 1)) >> BM_SHIFT
        s = plsc.cumsum(nb)
        ex = s - nb + carry
        sr[pl.ds(c * 16, 16)] = ex * BM
        plsc.addupdate_scatter(marks, [ex + nb], ones)
        carry = carry + jnp.sum(nb)
    nlive = carry

    bcarry = jnp.zeros((), jnp.int32)
    for c in range(NB // 16):
        m = marks[pl.ds(c * 16, 16)]
        cs = plsc.cumsum(m)
        bemap[pl.ds(c * 16, 16)] = cs + bcarry
        bcarry = bcarry + jnp.sum(m)

    @pl.when(wid == 0)
    def _():
        pltpu.sync_copy(bemap, be_hbm)
        stg[...] = jnp.full((16,), nlive, jnp.int32)
        pltpu.sync_copy(stg, nl_hbm)

    def pos_pass(j, _):
        ev = ec[pl.ds((wid * GW + j) * 16, 16)]
        grank = lr[pl.ds(j * 16, 16)]
        posc[pl.ds(j * 16, 16)] = plsc.load_gather(sr, [ev]) + grank
        return 0

    lax.fori_loop(0, GW, pos_pass, 0, unroll=4)

    pltpu.sync_copy(flat_hbm.at[pl.ds(wid * TW, TW)], tokrows)
    for k in range(TOP_K):
        for c in range(TW // 16):
            iks[k][pl.ds(c * 16, 16)] = plsc.load_gather(
                posc, [lane * TOP_K + c * 16 * TOP_K + k])
    for k in range(TOP_K):
        pltpu.async_copy(tokrows, xs_hbm.at[iks[k]], ss)
    for k in range(TOP_K):
        pltpu.make_async_copy(tokrows, xs_hbm.at[iks[k]], ss).wait()
    pltpu.sync_copy(posc, pos_hbm.at[pl.ds(ab, AW)])


def _gemm_body(blk_exp_ref, nlive_ref, x_ref, wgu_ref, wd_ref, y_ref):
    b = pl.program_id(0)

    @pl.when(b < nlive_ref[0])
    def _():
        x = x_ref[...]
        gu = lax.dot_general(x, wgu_ref[0], (((1,), (0,)), ((), ())),
                             preferred_element_type=jnp.float32)
        g = gu[:, :D_FF]
        u = gu[:, D_FF:]
        h = g * jax.nn.sigmoid(g) * u
        y_ref[...] = lax.dot_general(h, wd_ref[0], (((1,), (0,)), ((), ())),
                                     preferred_element_type=jnp.float32)


def _grouped_gemm(x_sorted, w_gate_up, w_down, blk_exp, nlive):
    def live_b(b, blk_exp_ref, nlive_ref):
        return jnp.minimum(b, nlive_ref[0] - 1)

    grid_spec = pltpu.PrefetchScalarGridSpec(
        num_scalar_prefetch=2,
        grid=(NB,),
        in_specs=[
            pl.BlockSpec((BM, D_MODEL), lambda b, be, nl: (live_b(b, be, nl), 0)),
            pl.BlockSpec((1, D_MODEL, 2 * D_FF),
                         lambda b, be, nl: (be[live_b(b, be, nl)], 0, 0)),
            pl.BlockSpec((1, D_FF, D_MODEL),
                         lambda b, be, nl: (be[live_b(b, be, nl)], 0, 0)),
        ],
        out_specs=pl.BlockSpec((BM, D_MODEL), lambda b, be, nl: (live_b(b, be, nl), 0)),
    )
    return pl.pallas_call(
        _gemm_body,
        grid_spec=grid_spec,
        out_shape=jax.ShapeDtypeStruct((PADN, D_MODEL), jnp.float32),
    )(blk_exp, nlive, x_sorted, w_gate_up, w_down)


@functools.partial(
    pl.kernel,
    out_type=jax.ShapeDtypeStruct((T, D_MODEL), jnp.float32),
    mesh=_SC_MESH,
    compiler_params=_SC_PARAMS,
    scratch_types=[
        pltpu.VMEM((AW,), jnp.int32),
        pltpu.VMEM((AW,), jnp.float32),
        [pltpu.VMEM((16,), jnp.int32) for _ in range(2)],
        [pltpu.VMEM((16, D_MODEL), jnp.float32) for _ in range(2)],
        pltpu.VMEM((TW, D_MODEL), jnp.float32),
        [pltpu.SemaphoreType.DMA for _ in range(2)],
    ],
)
def _combine(y_hbm, rw_hbm, pos_hbm, out_hbm, posc, wc, idx, rows, outr, sem):
    wid = lax.axis_index("s") * 2 + lax.axis_index("c")
    ab = wid * AW

    pltpu.sync_copy(pos_hbm.at[pl.ds(ab, AW)], posc)
    pltpu.sync_copy(rw_hbm.at[pl.ds(ab, AW)], wc)

    NBUF = 2

    def issue(p, j):
        idx[p][...] = posc[pl.ds(j * 16, 16)]
        pltpu.async_copy(y_hbm.at[idx[p]], rows[p], sem[p])

    def drain(p):
        pltpu.make_async_copy(y_hbm.at[idx[p]], rows[p], sem[p]).wait()

    def compute(p, j):
        wk = [plsc.load_gather(wc, [jnp.full((16,), j * 16 + k, jnp.int32)])
              for k in range(16)]
        for c in range(D_MODEL // 16):
            for lt in range(2):
                r = lambda k: rows[p][lt * 8 + k, pl.ds(c * 16, 16)]
                w = lambda k: wk[lt * 8 + k]
                acc_a = w(0) * r(0) + w(1) * r(1)
                acc_a = acc_a + w(2) * r(2) + w(3) * r(3)
                acc_b = w(4) * r(4) + w(5) * r(5)
                acc_b = acc_b + w(6) * r(6) + w(7) * r(7)
                outr[j * 2 + lt, pl.ds(c * 16, 16)] = acc_a + acc_b

    for p in range(NBUF):
        issue(p, p)

    def quad(i, _):
        j = i * NBUF
        for p in range(NBUF):
            drain(p)
            compute(p, j + p)

            @pl.when(j + p + NBUF < GW)
            def _():
                issue(p, j + p + NBUF)

        return 0

    lax.fori_loop(0, GW // NBUF, quad, 0)
    pltpu.sync_copy(outr, out_hbm.at[pl.ds(wid * TW, TW)])


def kernel(hidden_states, gate_w, w_gate_up, w_down):
    orig_shape = hidden_states.shape
    flat = hidden_states.reshape(T, D_MODEL)

    experts, rw = _router(flat, gate_w)
    x_sorted, pos, blk_exp, nlive = _shuffle(experts.reshape(A), flat)
    y = _grouped_gemm(x_sorted, w_gate_up, w_down, blk_exp, nlive)
    final = _combine(y, rw.reshape(A), pos)
    return final.reshape(orig_shape)


# speedup vs baseline: 1.2266x; 1.1601x over previous
import functools

import jax
import jax.numpy as jnp
from jax import lax
from jax.experimental import pallas as pl
from jax.experimental.pallas import tpu as pltpu
from jax.experimental.pallas import tpu_sc as plsc

E = 64
TOP_K = 8
D_MODEL = 768
D_FF = 384

T = 2048
A = T * TOP_K
BM = 256
BM_SHIFT = 8
NB = A // BM + E
PADN = NB * BM

TB = 256

NW = 32
AW = A // NW
GW = AW // 16
TW = T // NW
NG = A // 16

_SC_MESH = plsc.VectorSubcoreMesh(core_axis_name="c", subcore_axis_name="s",
                                  num_cores=2, num_subcores=16)
_SC_PARAMS = pltpu.CompilerParams(needs_layout_passes=False)


def _router_body(x_ref, gw_ref, exp_ref, rw_ref):
    x = x_ref[...]
    gw = gw_ref[...]
    logits = lax.dot_general(x, gw, (((1,), (1,)), ((), ())),
                             preferred_element_type=jnp.float32)
    m = jnp.max(logits, axis=-1, keepdims=True)
    p = jnp.exp(logits - m)
    p = p / jnp.sum(p, axis=-1, keepdims=True)

    iota = lax.broadcasted_iota(jnp.int32, (TB, E), 1)
    vals = []
    idxs = []
    for _ in range(TOP_K):
        mx = jnp.max(p, axis=-1, keepdims=True)
        idx = jnp.min(jnp.where(p == mx, iota, E), axis=-1, keepdims=True)
        vals.append(mx)
        idxs.append(idx)
        p = jnp.where(iota == idx, -1.0, p)
    v = jnp.concatenate(vals, axis=-1)
    ix = jnp.concatenate(idxs, axis=-1)
    v = v / jnp.sum(v, axis=-1, keepdims=True)
    exp_ref[...] = ix
    rw_ref[...] = v


def _router(flat, gate_w):
    return pl.pallas_call(
        _router_body,
        grid=(T // TB,),
        in_specs=[
            pl.BlockSpec((TB, D_MODEL), lambda i: (i, 0)),
            pl.BlockSpec((E, D_MODEL), lambda i: (0, 0)),
        ],
        out_specs=[
            pl.BlockSpec((TB, TOP_K), lambda i: (i, 0)),
            pl.BlockSpec((TB, TOP_K), lambda i: (i, 0)),
        ],
        out_shape=[
            jax.ShapeDtypeStruct((T, TOP_K), jnp.int32),
            jax.ShapeDtypeStruct((T, TOP_K), jnp.float32),
        ],
    )(flat, gate_w)


@functools.partial(
    pl.kernel,
    out_type=[
        jax.ShapeDtypeStruct((PADN, D_MODEL), jnp.float32),
        jax.ShapeDtypeStruct((A,), jnp.int32),
        jax.ShapeDtypeStruct((NB,), jnp.int32),
        jax.ShapeDtypeStruct((16,), jnp.int32),
    ],
    mesh=_SC_MESH,
    compiler_params=_SC_PARAMS,
    scratch_types=[
        pltpu.VMEM((A,), jnp.int32),
        pltpu.VMEM((E,), jnp.int32),
        pltpu.VMEM((AW,), jnp.int32),
        pltpu.VMEM((AW,), jnp.int32),
        pltpu.VMEM((E,), jnp.int32),
        pltpu.VMEM((NB + 16,), jnp.int32),
        pltpu.VMEM((NB,), jnp.int32),
        pltpu.VMEM((TW, D_MODEL), jnp.float32),
        [pltpu.VMEM((TW,), jnp.int32) for _ in range(TOP_K)],
        pltpu.VMEM((16,), jnp.int32),
        pltpu.SemaphoreType.DMA,
    ],
)
def _shuffle(e_hbm, flat_hbm, xs_hbm, pos_hbm, be_hbm, nl_hbm,
             ec, hist, lr, posc, sr, marks, bemap, tokrows, iks, stg, ss):
    wid = lax.axis_index("s") * 2 + lax.axis_index("c")
    ab = wid * AW

    pltpu.sync_copy(e_hbm, ec)
    for c in range(E // 16):
        hist[pl.ds(c * 16, 16)] = jnp.zeros((16,), jnp.int32)

    ones = jnp.ones((16,), jnp.int32)
    lane = lax.iota(jnp.int32, 16)

    def hist_pass(j, _):
        ev = ec[pl.ds(j * 16, 16)]
        cnt, _last = plsc.scan_count(ev)
        prior = plsc.load_gather(hist, [ev])

        @pl.when((j >= wid * GW) & (j < (wid + 1) * GW))
        def _():
            lr[pl.ds((j - wid * GW) * 16, 16)] = prior + cnt - 1

        plsc.addupdate_scatter(hist, [ev], ones)
        return 0

    lax.fori_loop(0, NG, hist_pass, 0, unroll=4)

    for c in range((NB + 16) // 16):
        marks[pl.ds(c * 16, 16)] = jnp.zeros((16,), jnp.int32)
    carry = jnp.zeros((), jnp.int32)
    for c in range(E // 16):
        h = hist[pl.ds(c * 16, 16)]
        nb = (h + (BM - 1)) >> BM_SHIFT
        s = plsc.cumsum(nb)
        ex = s - nb + carry
        sr[pl.ds(c * 16, 16)] = ex * BM
        plsc.addupdate_scatter(marks, [ex + nb], ones)
        carry = carry + jnp.sum(nb)
    nlive = carry

    bcarry = jnp.zeros((), jnp.int32)
    for c in range(NB // 16):
        m = marks[pl.ds(c * 16, 16)]
        cs = plsc.cumsum(m)
        bemap[pl.ds(c * 16, 16)] = cs + bcarry
        bcarry = bcarry + jnp.sum(m)

    @pl.when(wid == 0)
    def _():
        pltpu.sync_copy(bemap, be_hbm)
        stg[...] = jnp.full((16,), nlive, jnp.int32)
        pltpu.sync_copy(stg, nl_hbm)

    def pos_pass(j, _):
        ev = ec[pl.ds((wid * GW + j) * 16, 16)]
        grank = lr[pl.ds(j * 16, 16)]
        posc[pl.ds(j * 16, 16)] = plsc.load_gather(sr, [ev]) + grank
        return 0

    lax.fori_loop(0, GW, pos_pass, 0, unroll=4)

    pltpu.sync_copy(flat_hbm.at[pl.ds(wid * TW, TW)], tokrows)
    for k in range(TOP_K):
        for c in range(TW // 16):
            iks[k][pl.ds(c * 16, 16)] = plsc.load_gather(
                posc, [lane * TOP_K + c * 16 * TOP_K + k])
    for k in range(TOP_K):
        pltpu.async_copy(tokrows, xs_hbm.at[iks[k]], ss)
    for k in range(TOP_K):
        pltpu.make_async_copy(tokrows, xs_hbm.at[iks[k]], ss).wait()
    pltpu.sync_copy(posc, pos_hbm.at[pl.ds(ab, AW)])


def _gemm_body(blk_exp_ref, nlive_ref, x_ref, wgu_ref, wd_ref, y_ref):
    b = pl.program_id(0)

    @pl.when(b < nlive_ref[0])
    def _():
        x = x_ref[...]
        gu = lax.dot_general(x, wgu_ref[0], (((1,), (0,)), ((), ())),
                             preferred_element_type=jnp.float32)
        g = gu[:, :D_FF]
        u = gu[:, D_FF:]
        h = g * jax.nn.sigmoid(g) * u
        y_ref[...] = lax.dot_general(h, wd_ref[0], (((1,), (0,)), ((), ())),
                                     preferred_element_type=jnp.float32)


def _grouped_gemm(x_sorted, w_gate_up, w_down, blk_exp, nlive):
    def live_b(b, blk_exp_ref, nlive_ref):
        return jnp.minimum(b, nlive_ref[0] - 1)

    grid_spec = pltpu.PrefetchScalarGridSpec(
        num_scalar_prefetch=2,
        grid=(NB,),
        in_specs=[
            pl.BlockSpec((BM, D_MODEL), lambda b, be, nl: (live_b(b, be, nl), 0)),
            pl.BlockSpec((1, D_MODEL, 2 * D_FF),
                         lambda b, be, nl: (be[live_b(b, be, nl)], 0, 0)),
            pl.BlockSpec((1, D_FF, D_MODEL),
                         lambda b, be, nl: (be[live_b(b, be, nl)], 0, 0)),
        ],
        out_specs=pl.BlockSpec((BM, D_MODEL), lambda b, be, nl: (live_b(b, be, nl), 0)),
    )
    return pl.pallas_call(
        _gemm_body,
        grid_spec=grid_spec,
        out_shape=jax.ShapeDtypeStruct((PADN, D_MODEL), jnp.float32),
    )(blk_exp, nlive, x_sorted, w_gate_up, w_down)


@functools.partial(
    pl.kernel,
    out_type=jax.ShapeDtypeStruct((T, D_MODEL), jnp.float32),
    mesh=_SC_MESH,
    compiler_params=_SC_PARAMS,
    scratch_types=[
        pltpu.VMEM((AW,), jnp.int32),
        pltpu.VMEM((AW,), jnp.float32),
        [pltpu.VMEM((16,), jnp.int32) for _ in range(2)],
        [pltpu.VMEM((16, D_MODEL), jnp.float32) for _ in range(2)],
        pltpu.VMEM((TW, D_MODEL), jnp.float32),
        [pltpu.SemaphoreType.DMA for _ in range(2)],
    ],
)
def _combine(y_hbm, rw_hbm, pos_hbm, out_hbm, posc, wc, idx, rows, outr, sem):
    wid = lax.axis_index("s") * 2 + lax.axis_index("c")
    ab = wid * AW

    pltpu.sync_copy(pos_hbm.at[pl.ds(ab, AW)], posc)
    pltpu.sync_copy(rw_hbm.at[pl.ds(ab, AW)], wc)

    NBUF = 2

    def issue(p, j):
        idx[p][...] = posc[pl.ds(j * 16, 16)]
        pltpu.async_copy(y_hbm.at[idx[p]], rows[p], sem[p])

    def drain(p):
        pltpu.make_async_copy(y_hbm.at[idx[p]], rows[p], sem[p]).wait()

    def compute(p, j):
        wk = [plsc.load_gather(wc, [jnp.full((16,), j * 16 + k, jnp.int32)])
              for k in range(16)]

        def cbody(c, _):
            for lt in range(2):
                r = lambda k: rows[p][lt * 8 + k, pl.ds(c * 16, 16)]
                w = lambda k: wk[lt * 8 + k]
                acc_a = w(0) * r(0) + w(1) * r(1)
                acc_a = acc_a + w(2) * r(2) + w(3) * r(3)
                acc_b = w(4) * r(4) + w(5) * r(5)
                acc_b = acc_b + w(6) * r(6) + w(7) * r(7)
                outr[j * 2 + lt, pl.ds(c * 16, 16)] = acc_a + acc_b
            return 0

        lax.fori_loop(0, D_MODEL // 16, cbody, 0, unroll=4)

    for p in range(NBUF):
        issue(p, p)

    def quad(i, _):
        j = i * NBUF
        for p in range(NBUF):
            drain(p)
            compute(p, j + p)

            @pl.when(j + p + NBUF < GW)
            def _():
                issue(p, j + p + NBUF)

        return 0

    lax.fori_loop(0, GW // NBUF, quad, 0)
    pltpu.sync_copy(outr, out_hbm.at[pl.ds(wid * TW, TW)])


def kernel(hidden_states, gate_w, w_gate_up, w_down):
    orig_shape = hidden_states.shape
    flat = hidden_states.reshape(T, D_MODEL)

    experts, rw = _router(flat, gate_w)
    x_sorted, pos, blk_exp, nlive = _shuffle(experts.reshape(A), flat)
    y = _grouped_gemm(x_sorted, w_gate_up, w_down, blk_exp, nlive)
    final = _combine(y, rw.reshape(A), pos)
    return final.reshape(orig_shape)


# speedup vs baseline: 1.2277x; 1.0009x over previous
import functools

import jax
import jax.numpy as jnp
from jax import lax
from jax.experimental import pallas as pl
from jax.experimental.pallas import tpu as pltpu
from jax.experimental.pallas import tpu_sc as plsc

E = 64
TOP_K = 8
D_MODEL = 768
D_FF = 384

T = 2048
A = T * TOP_K
BM = 256
BM_SHIFT = 8
NB = A // BM + E
PADN = NB * BM

TB = 256

NW = 32
AW = A // NW
GW = AW // 16
TW = T // NW
NG = A // 16

_SC_MESH = plsc.VectorSubcoreMesh(core_axis_name="c", subcore_axis_name="s",
                                  num_cores=2, num_subcores=16)
_SC_PARAMS = pltpu.CompilerParams(needs_layout_passes=False)


def _router_body(x_ref, gw_ref, exp_ref, rw_ref):
    x = x_ref[...]
    gw = gw_ref[...]
    logits = lax.dot_general(x, gw, (((1,), (1,)), ((), ())),
                             preferred_element_type=jnp.float32)
    m = jnp.max(logits, axis=-1, keepdims=True)
    p = jnp.exp(logits - m)
    p = p / jnp.sum(p, axis=-1, keepdims=True)

    iota = lax.broadcasted_iota(jnp.int32, (TB, E), 1)
    vals = []
    idxs = []
    for _ in range(TOP_K):
        mx = jnp.max(p, axis=-1, keepdims=True)
        idx = jnp.min(jnp.where(p == mx, iota, E), axis=-1, keepdims=True)
        vals.append(mx)
        idxs.append(idx)
        p = jnp.where(iota == idx, -1.0, p)
    v = jnp.concatenate(vals, axis=-1)
    ix = jnp.concatenate(idxs, axis=-1)
    v = v / jnp.sum(v, axis=-1, keepdims=True)
    exp_ref[...] = ix
    rw_ref[...] = v


def _router(flat, gate_w):
    return pl.pallas_call(
        _router_body,
        grid=(T // TB,),
        in_specs=[
            pl.BlockSpec((TB, D_MODEL), lambda i: (i, 0)),
            pl.BlockSpec((E, D_MODEL), lambda i: (0, 0)),
        ],
        out_specs=[
            pl.BlockSpec((TB, TOP_K), lambda i: (i, 0)),
            pl.BlockSpec((TB, TOP_K), lambda i: (i, 0)),
        ],
        out_shape=[
            jax.ShapeDtypeStruct((T, TOP_K), jnp.int32),
            jax.ShapeDtypeStruct((T, TOP_K), jnp.float32),
        ],
    )(flat, gate_w)


@functools.partial(
    pl.kernel,
    out_type=[
        jax.ShapeDtypeStruct((PADN, D_MODEL), jnp.float32),
        jax.ShapeDtypeStruct((A,), jnp.int32),
        jax.ShapeDtypeStruct((NB,), jnp.int32),
        jax.ShapeDtypeStruct((16,), jnp.int32),
    ],
    mesh=_SC_MESH,
    compiler_params=_SC_PARAMS,
    scratch_types=[
        pltpu.VMEM((A,), jnp.int32),
        pltpu.VMEM((E,), jnp.int32),
        pltpu.VMEM((AW,), jnp.int32),
        pltpu.VMEM((AW,), jnp.int32),
        pltpu.VMEM((E,), jnp.int32),
        pltpu.VMEM((NB + 16,), jnp.int32),
        pltpu.VMEM((NB,), jnp.int32),
        pltpu.VMEM((TW, D_MODEL), jnp.float32),
        [pltpu.VMEM((TW,), jnp.int32) for _ in range(TOP_K)],
        pltpu.VMEM((16,), jnp.int32),
        pltpu.SemaphoreType.DMA,
    ],
)
def _shuffle(e_hbm, flat_hbm, xs_hbm, pos_hbm, be_hbm, nl_hbm,
             ec, hist, lr, posc, sr, marks, bemap, tokrows, iks, stg, ss):
    wid = lax.axis_index("s") * 2 + lax.axis_index("c")
    ab = wid * AW

    pltpu.sync_copy(e_hbm, ec)
    for c in range(E // 16):
        hist[pl.ds(c * 16, 16)] = jnp.zeros((16,), jnp.int32)

    ones = jnp.ones((16,), jnp.int32)
    lane = lax.iota(jnp.int32, 16)

    def hist_pass(j, _):
        ev = ec[pl.ds(j * 16, 16)]
        cnt, _last = plsc.scan_count(ev)
        prior = plsc.load_gather(hist, [ev])

        @pl.when((j >= wid * GW) & (j < (wid + 1) * GW))
        def _():
            lr[pl.ds((j - wid * GW) * 16, 16)] = prior + cnt - 1

        plsc.addupdate_scatter(hist, [ev], ones)
        return 0

    lax.fori_loop(0, NG, hist_pass, 0, unroll=4)

    for c in range((NB + 16) // 16):
        marks[pl.ds(c * 16, 16)] = jnp.zeros((16,), jnp.int32)
    carry = jnp.zeros((), jnp.int32)
    for c in range(E // 16):
        h = hist[pl.ds(c * 16, 16)]
        nb = (h + (BM - 1)) >> BM_SHIFT
        s = plsc.cumsum(nb)
        ex = s - nb + carry
        sr[pl.ds(c * 16, 16)] = ex * BM
        plsc.addupdate_scatter(marks, [ex + nb], ones)
        carry = carry + jnp.sum(nb)
    nlive = carry

    bcarry = jnp.zeros((), jnp.int32)
    for c in range(NB // 16):
        m = marks[pl.ds(c * 16, 16)]
        cs = plsc.cumsum(m)
        bemap[pl.ds(c * 16, 16)] = cs + bcarry
        bcarry = bcarry + jnp.sum(m)

    @pl.when(wid == 0)
    def _():
        pltpu.sync_copy(bemap, be_hbm)
        stg[...] = jnp.full((16,), nlive, jnp.int32)
        pltpu.sync_copy(stg, nl_hbm)

    def pos_pass(j, _):
        ev = ec[pl.ds((wid * GW + j) * 16, 16)]
        grank = lr[pl.ds(j * 16, 16)]
        posc[pl.ds(j * 16, 16)] = plsc.load_gather(sr, [ev]) + grank
        return 0

    lax.fori_loop(0, GW, pos_pass, 0, unroll=4)

    pltpu.sync_copy(flat_hbm.at[pl.ds(wid * TW, TW)], tokrows)
    for k in range(TOP_K):
        for c in range(TW // 16):
            iks[k][pl.ds(c * 16, 16)] = plsc.load_gather(
                posc, [lane * TOP_K + c * 16 * TOP_K + k])
    for k in range(TOP_K):
        pltpu.async_copy(tokrows, xs_hbm.at[iks[k]], ss)
    for k in range(TOP_K):
        pltpu.make_async_copy(tokrows, xs_hbm.at[iks[k]], ss).wait()
    pltpu.sync_copy(posc, pos_hbm.at[pl.ds(ab, AW)])


def _gemm_body(blk_exp_ref, nlive_ref, x_ref, wgu_ref, wd_ref, y_ref):
    b = pl.program_id(0)

    @pl.when(b < nlive_ref[0])
    def _():
        x = x_ref[...]
        gu = lax.dot_general(x, wgu_ref[0], (((1,), (0,)), ((), ())),
                             preferred_element_type=jnp.float32)
        g = gu[:, :D_FF]
        u = gu[:, D_FF:]
        h = g * jax.nn.sigmoid(g) * u
        y_ref[...] = lax.dot_general(h, wd_ref[0], (((1,), (0,)), ((), ())),
                                     preferred_element_type=jnp.float32)


def _grouped_gemm(x_sorted, w_gate_up, w_down, blk_exp, nlive):
    def live_b(b, blk_exp_ref, nlive_ref):
        return jnp.minimum(b, nlive_ref[0] - 1)

    grid_spec = pltpu.PrefetchScalarGridSpec(
        num_scalar_prefetch=2,
        grid=(NB,),
        in_specs=[
            pl.BlockSpec((BM, D_MODEL), lambda b, be, nl: (live_b(b, be, nl), 0)),
            pl.BlockSpec((1, D_MODEL, 2 * D_FF),
                         lambda b, be, nl: (be[live_b(b, be, nl)], 0, 0)),
            pl.BlockSpec((1, D_FF, D_MODEL),
                         lambda b, be, nl: (be[live_b(b, be, nl)], 0, 0)),
        ],
        out_specs=pl.BlockSpec((BM, D_MODEL), lambda b, be, nl: (live_b(b, be, nl), 0)),
    )
    return pl.pallas_call(
        _gemm_body,
        grid_spec=grid_spec,
        out_shape=jax.ShapeDtypeStruct((PADN, D_MODEL), jnp.float32),
    )(blk_exp, nlive, x_sorted, w_gate_up, w_down)


@functools.partial(
    pl.kernel,
    out_type=jax.ShapeDtypeStruct((T, D_MODEL), jnp.float32),
    mesh=_SC_MESH,
    compiler_params=_SC_PARAMS,
    scratch_types=[
        pltpu.VMEM((AW,), jnp.int32),
        pltpu.VMEM((AW,), jnp.float32),
        [pltpu.VMEM((16,), jnp.int32) for _ in range(2)],
        [pltpu.VMEM((16, D_MODEL), jnp.float32) for _ in range(2)],
        pltpu.VMEM((TW, D_MODEL), jnp.float32),
        [pltpu.SemaphoreType.DMA for _ in range(2)],
    ],
)
def _combine(y_hbm, rw_hbm, pos_hbm, out_hbm, posc, wc, idx, rows, outr, sem):
    wid = lax.axis_index("s") * 2 + lax.axis_index("c")
    ab = wid * AW

    pltpu.sync_copy(pos_hbm.at[pl.ds(ab, AW)], posc)
    pltpu.sync_copy(rw_hbm.at[pl.ds(ab, AW)], wc)

    NBUF = 2

    def issue(p, j):
        idx[p][...] = posc[pl.ds(j * 16, 16)]
        pltpu.async_copy(y_hbm.at[idx[p]], rows[p], sem[p])

    def drain(p):
        pltpu.make_async_copy(y_hbm.at[idx[p]], rows[p], sem[p]).wait()

    def compute(p, j):
        wk = [plsc.load_gather(wc, [jnp.full((16,), j * 16 + k, jnp.int32)])
              for k in range(16)]

        def cbody(c, _):
            for lt in range(2):
                r = lambda k: rows[p][lt * 8 + k, pl.ds(c * 16, 16)]
                w = lambda k: wk[lt * 8 + k]
                acc_a = w(0) * r(0) + w(1) * r(1)
                acc_a = acc_a + w(2) * r(2) + w(3) * r(3)
                acc_b = w(4) * r(4) + w(5) * r(5)
                acc_b = acc_b + w(6) * r(6) + w(7) * r(7)
                outr[j * 2 + lt, pl.ds(c * 16, 16)] = acc_a + acc_b
            return 0

        lax.fori_loop(0, D_MODEL // 16, cbody, 0, unroll=8)

    for p in range(NBUF):
        issue(p, p)

    def quad(i, _):
        j = i * NBUF
        for p in range(NBUF):
            drain(p)
            compute(p, j + p)

            @pl.when(j + p + NBUF < GW)
            def _():
                issue(p, j + p + NBUF)

        return 0

    lax.fori_loop(0, GW // NBUF, quad, 0)
    pltpu.sync_copy(outr, out_hbm.at[pl.ds(wid * TW, TW)])


def kernel(hidden_states, gate_w, w_gate_up, w_down):
    orig_shape = hidden_states.shape
    flat = hidden_states.reshape(T, D_MODEL)

    experts, rw = _router(flat, gate_w)
    x_sorted, pos, blk_exp, nlive = _shuffle(experts.reshape(A), flat)
    y = _grouped_gemm(x_sorted, w_gate_up, w_down, blk_exp, nlive)
    final = _combine(y, rw.reshape(A), pos)
    return final.reshape(orig_shape)


# speedup vs baseline: 1.2295x; 1.0015x over previous
import functools

import jax
import jax.numpy as jnp
from jax import lax
from jax.experimental import pallas as pl
from jax.experimental.pallas import tpu as pltpu
from jax.experimental.pallas import tpu_sc as plsc

E = 64
TOP_K = 8
D_MODEL = 768
D_FF = 384

T = 2048
A = T * TOP_K
BM = 256
BM_SHIFT = 8
NB = A // BM + E
PADN = NB * BM

TB = 256

NW = 32
AW = A // NW
GW = AW // 16
TW = T // NW
NG = A // 16

_SC_MESH = plsc.VectorSubcoreMesh(core_axis_name="c", subcore_axis_name="s",
                                  num_cores=2, num_subcores=16)
_SC_PARAMS = pltpu.CompilerParams(needs_layout_passes=False)


def _router_body(x_ref, gw_ref, exp_ref, rw_ref):
    x = x_ref[...]
    gw = gw_ref[...]
    logits = lax.dot_general(x, gw, (((1,), (1,)), ((), ())),
                             preferred_element_type=jnp.float32)
    m = jnp.max(logits, axis=-1, keepdims=True)
    p = jnp.exp(logits - m)
    p = p / jnp.sum(p, axis=-1, keepdims=True)

    iota = lax.broadcasted_iota(jnp.int32, (TB, E), 1)
    vals = []
    idxs = []
    for _ in range(TOP_K):
        mx = jnp.max(p, axis=-1, keepdims=True)
        idx = jnp.min(jnp.where(p == mx, iota, E), axis=-1, keepdims=True)
        vals.append(mx)
        idxs.append(idx)
        p = jnp.where(iota == idx, -1.0, p)
    v = jnp.concatenate(vals, axis=-1)
    ix = jnp.concatenate(idxs, axis=-1)
    v = v / jnp.sum(v, axis=-1, keepdims=True)
    exp_ref[...] = ix
    rw_ref[...] = v


def _router(flat, gate_w):
    return pl.pallas_call(
        _router_body,
        grid=(T // TB,),
        in_specs=[
            pl.BlockSpec((TB, D_MODEL), lambda i: (i, 0)),
            pl.BlockSpec((E, D_MODEL), lambda i: (0, 0)),
        ],
        out_specs=[
            pl.BlockSpec((TB, TOP_K), lambda i: (i, 0)),
            pl.BlockSpec((TB, TOP_K), lambda i: (i, 0)),
        ],
        out_shape=[
            jax.ShapeDtypeStruct((T, TOP_K), jnp.int32),
            jax.ShapeDtypeStruct((T, TOP_K), jnp.float32),
        ],
    )(flat, gate_w)


@functools.partial(
    pl.kernel,
    out_type=[
        jax.ShapeDtypeStruct((PADN, D_MODEL), jnp.float32),
        jax.ShapeDtypeStruct((A,), jnp.int32),
        jax.ShapeDtypeStruct((NB,), jnp.int32),
        jax.ShapeDtypeStruct((16,), jnp.int32),
    ],
    mesh=_SC_MESH,
    compiler_params=_SC_PARAMS,
    scratch_types=[
        pltpu.VMEM((A,), jnp.int32),
        pltpu.VMEM((E,), jnp.int32),
        pltpu.VMEM((AW,), jnp.int32),
        pltpu.VMEM((AW,), jnp.int32),
        pltpu.VMEM((E,), jnp.int32),
        pltpu.VMEM((NB + 16,), jnp.int32),
        pltpu.VMEM((NB,), jnp.int32),
        pltpu.VMEM((TW, D_MODEL), jnp.float32),
        [pltpu.VMEM((TW,), jnp.int32) for _ in range(TOP_K)],
        pltpu.VMEM((16,), jnp.int32),
        pltpu.SemaphoreType.DMA,
    ],
)
def _shuffle(e_hbm, flat_hbm, xs_hbm, pos_hbm, be_hbm, nl_hbm,
             ec, hist, lr, posc, sr, marks, bemap, tokrows, iks, stg, ss):
    wid = lax.axis_index("s") * 2 + lax.axis_index("c")
    ab = wid * AW

    pltpu.sync_copy(e_hbm, ec)
    for c in range(E // 16):
        hist[pl.ds(c * 16, 16)] = jnp.zeros((16,), jnp.int32)

    ones = jnp.ones((16,), jnp.int32)
    lane = lax.iota(jnp.int32, 16)

    def hist_pass(j, _):
        ev = ec[pl.ds(j * 16, 16)]
        cnt, _last = plsc.scan_count(ev)
        prior = plsc.load_gather(hist, [ev])

        @pl.when((j >= wid * GW) & (j < (wid + 1) * GW))
        def _():
            lr[pl.ds((j - wid * GW) * 16, 16)] = prior + cnt - 1

        plsc.addupdate_scatter(hist, [ev], ones)
        return 0

    lax.fori_loop(0, NG, hist_pass, 0, unroll=4)

    for c in range((NB + 16) // 16):
        marks[pl.ds(c * 16, 16)] = jnp.zeros((16,), jnp.int32)
    carry = jnp.zeros((), jnp.int32)
    for c in range(E // 16):
        h = hist[pl.ds(c * 16, 16)]
        nb = (h + (BM - 1)) >> BM_SHIFT
        s = plsc.cumsum(nb)
        ex = s - nb + carry
        sr[pl.ds(c * 16, 16)] = ex * BM
        plsc.addupdate_scatter(marks, [ex + nb], ones)
        carry = carry + jnp.sum(nb)
    nlive = carry

    bcarry = jnp.zeros((), jnp.int32)
    for c in range(NB // 16):
        m = marks[pl.ds(c * 16, 16)]
        cs = plsc.cumsum(m)
        bemap[pl.ds(c * 16, 16)] = cs + bcarry
        bcarry = bcarry + jnp.sum(m)

    @pl.when(wid == 0)
    def _():
        pltpu.sync_copy(bemap, be_hbm)
        stg[...] = jnp.full((16,), nlive, jnp.int32)
        pltpu.sync_copy(stg, nl_hbm)

    def pos_pass(j, _):
        ev = ec[pl.ds((wid * GW + j) * 16, 16)]
        grank = lr[pl.ds(j * 16, 16)]
        posc[pl.ds(j * 16, 16)] = plsc.load_gather(sr, [ev]) + grank
        return 0

    lax.fori_loop(0, GW, pos_pass, 0, unroll=4)

    pltpu.sync_copy(flat_hbm.at[pl.ds(wid * TW, TW)], tokrows)
    for k in range(TOP_K):
        for c in range(TW // 16):
            iks[k][pl.ds(c * 16, 16)] = plsc.load_gather(
                posc, [lane * TOP_K + c * 16 * TOP_K + k])
    for k in range(TOP_K):
        pltpu.async_copy(tokrows, xs_hbm.at[iks[k]], ss)
    for k in range(TOP_K):
        pltpu.make_async_copy(tokrows, xs_hbm.at[iks[k]], ss).wait()
    pltpu.sync_copy(posc, pos_hbm.at[pl.ds(ab, AW)])


def _gemm_body(blk_exp_ref, nlive_ref, x_ref, wgu_ref, wd_ref, y_ref):
    b = pl.program_id(0)

    @pl.when(b < nlive_ref[0])
    def _():
        x = x_ref[...]
        gu = lax.dot_general(x, wgu_ref[0], (((1,), (0,)), ((), ())),
                             preferred_element_type=jnp.float32)
        g = gu[:, :D_FF]
        u = gu[:, D_FF:]
        h = g * jax.nn.sigmoid(g) * u
        y_ref[...] = lax.dot_general(h, wd_ref[0], (((1,), (0,)), ((), ())),
                                     preferred_element_type=jnp.float32)


def _grouped_gemm(x_sorted, w_gate_up, w_down, blk_exp, nlive):
    def live_b(b, blk_exp_ref, nlive_ref):
        return jnp.minimum(b, nlive_ref[0] - 1)

    grid_spec = pltpu.PrefetchScalarGridSpec(
        num_scalar_prefetch=2,
        grid=(NB,),
        in_specs=[
            pl.BlockSpec((BM, D_MODEL), lambda b, be, nl: (live_b(b, be, nl), 0)),
            pl.BlockSpec((1, D_MODEL, 2 * D_FF),
                         lambda b, be, nl: (be[live_b(b, be, nl)], 0, 0)),
            pl.BlockSpec((1, D_FF, D_MODEL),
                         lambda b, be, nl: (be[live_b(b, be, nl)], 0, 0)),
        ],
        out_specs=pl.BlockSpec((BM, D_MODEL), lambda b, be, nl: (live_b(b, be, nl), 0)),
    )
    return pl.pallas_call(
        _gemm_body,
        grid_spec=grid_spec,
        out_shape=jax.ShapeDtypeStruct((PADN, D_MODEL), jnp.float32),
    )(blk_exp, nlive, x_sorted, w_gate_up, w_down)


@functools.partial(
    pl.kernel,
    out_type=jax.ShapeDtypeStruct((T, D_MODEL), jnp.float32),
    mesh=_SC_MESH,
    compiler_params=_SC_PARAMS,
    scratch_types=[
        pltpu.VMEM((AW,), jnp.int32),
        pltpu.VMEM((AW,), jnp.float32),
        [pltpu.VMEM((16,), jnp.int32) for _ in range(4)],
        [pltpu.VMEM((16, D_MODEL), jnp.float32) for _ in range(4)],
        pltpu.VMEM((TW, D_MODEL), jnp.float32),
        [pltpu.SemaphoreType.DMA for _ in range(4)],
    ],
)
def _combine(y_hbm, rw_hbm, pos_hbm, out_hbm, posc, wc, idx, rows, outr, sem):
    wid = lax.axis_index("s") * 2 + lax.axis_index("c")
    ab = wid * AW

    pltpu.sync_copy(pos_hbm.at[pl.ds(ab, AW)], posc)
    pltpu.sync_copy(rw_hbm.at[pl.ds(ab, AW)], wc)

    NBUF = 4

    def issue(p, j):
        idx[p][...] = posc[pl.ds(j * 16, 16)]
        pltpu.async_copy(y_hbm.at[idx[p]], rows[p], sem[p])

    def drain(p):
        pltpu.make_async_copy(y_hbm.at[idx[p]], rows[p], sem[p]).wait()

    def compute(p, j):
        wk = [plsc.load_gather(wc, [jnp.full((16,), j * 16 + k, jnp.int32)])
              for k in range(16)]

        def cbody(c, _):
            for lt in range(2):
                r = lambda k: rows[p][lt * 8 + k, pl.ds(c * 16, 16)]
                w = lambda k: wk[lt * 8 + k]
                acc_a = w(0) * r(0) + w(1) * r(1)
                acc_a = acc_a + w(2) * r(2) + w(3) * r(3)
                acc_b = w(4) * r(4) + w(5) * r(5)
                acc_b = acc_b + w(6) * r(6) + w(7) * r(7)
                outr[j * 2 + lt, pl.ds(c * 16, 16)] = acc_a + acc_b
            return 0

        lax.fori_loop(0, D_MODEL // 16, cbody, 0, unroll=8)

    for p in range(NBUF):
        issue(p, p)

    def quad(i, _):
        j = i * NBUF
        for p in range(NBUF):
            drain(p)
            compute(p, j + p)

            @pl.when(j + p + NBUF < GW)
            def _():
                issue(p, j + p + NBUF)

        return 0

    lax.fori_loop(0, GW // NBUF, quad, 0)
    pltpu.sync_copy(outr, out_hbm.at[pl.ds(wid * TW, TW)])


def kernel(hidden_states, gate_w, w_gate_up, w_down):
    orig_shape = hidden_states.shape
    flat = hidden_states.reshape(T, D_MODEL)

    experts, rw = _router(flat, gate_w)
    x_sorted, pos, blk_exp, nlive = _shuffle(experts.reshape(A), flat)
    y = _grouped_gemm(x_sorted, w_gate_up, w_down, blk_exp, nlive)
    final = _combine(y, rw.reshape(A), pos)
    return final.reshape(orig_shape)
